# baseline (speedup 1.0000x reference)
import functools

import jax
import jax.numpy as jnp
from jax import lax
from jax.experimental import pallas as pl
from jax.experimental.pallas import tpu as pltpu
from jax.experimental.pallas import tpu_sc as plsc

TOTAL = 16384 * 200
WLEN = 100000
NW = 32
PER_TILE = TOTAL // NW
CHUNK = 4096
NCHUNK = PER_TILE // CHUNK
LANES = 16
G = 16


def _sc_body(x_hbm, idx_hbm, w_hbm, out_hbm,
             w_v, idx_v, x_v, out_v, w_sem, in_sem, out_sem):
    wid = lax.axis_index("s") * 2 + lax.axis_index("c")
    tile_base = wid * PER_TILE

    def start_in(ci):
        b = ci % 2
        base = tile_base + ci * CHUNK
        pltpu.make_async_copy(
            idx_hbm.at[pl.ds(base, CHUNK)], idx_v.at[b], in_sem.at[b]).start()
        pltpu.make_async_copy(
            x_hbm.at[pl.ds(base, CHUNK)], x_v.at[b], in_sem.at[b]).start()

    def wait_in(ci):
        b = ci % 2
        base = tile_base + ci * CHUNK
        pltpu.make_async_copy(
            idx_hbm.at[pl.ds(base, CHUNK)], idx_v.at[b], in_sem.at[b]).wait()
        pltpu.make_async_copy(
            x_hbm.at[pl.ds(base, CHUNK)], x_v.at[b], in_sem.at[b]).wait()

    def start_out(ci):
        b = ci % 2
        base = tile_base + ci * CHUNK
        pltpu.make_async_copy(
            out_v.at[b], out_hbm.at[pl.ds(base, CHUNK)], out_sem.at[b]).start()

    def wait_out(ci):
        b = ci % 2
        base = tile_base + ci * CHUNK
        pltpu.make_async_copy(
            out_v.at[b], out_hbm.at[pl.ds(base, CHUNK)], out_sem.at[b]).wait()

    pltpu.make_async_copy(w_hbm, w_v, w_sem).start()
    start_in(0)
    pltpu.make_async_copy(w_hbm, w_v, w_sem).wait()

    for ci in range(NCHUNK):
        b = ci % 2
        if ci + 1 < NCHUNK:
            start_in(ci + 1)
        wait_in(ci)
        if ci >= 2:
            wait_out(ci - 2)

        def inner(i, _):
            base = i * (G * LANES)
            ivs = [idx_v[b, pl.ds(base + k * LANES, LANES)] for k in range(G)]
            ws = [plsc.load_gather(w_v, [iv]) for iv in ivs]
            xs = [x_v[b, pl.ds(base + k * LANES, LANES)] for k in range(G)]
            for k in range(G):
                out_v[b, pl.ds(base + k * LANES, LANES)] = ws[k] * xs[k]
            return 0

        lax.fori_loop(0, CHUNK // (G * LANES), inner, 0)
        start_out(ci)

    wait_out(NCHUNK - 2)
    wait_out(NCHUNK - 1)


@jax.jit
def _run(x_flat, idx_flat, weight):
    mesh = plsc.VectorSubcoreMesh(core_axis_name="c", subcore_axis_name="s")
    f = functools.partial(
        pl.kernel,
        mesh=mesh,
        out_type=jax.ShapeDtypeStruct((TOTAL,), jnp.float32),
        scratch_types=[
            pltpu.VMEM((WLEN,), jnp.float32),
            pltpu.VMEM((2, CHUNK), jnp.int32),
            pltpu.VMEM((2, CHUNK), jnp.float32),
            pltpu.VMEM((2, CHUNK), jnp.float32),
            pltpu.SemaphoreType.DMA,
            pltpu.SemaphoreType.DMA((2,)),
            pltpu.SemaphoreType.DMA((2,)),
        ],
        compiler_params=pltpu.CompilerParams(needs_layout_passes=False),
    )(_sc_body)
    return f(x_flat, idx_flat, weight)


_P0, _P1, _SUB, _LN = 25, 128, 8, 128


def _to_phys(a):
    return a.T.reshape(_P0, _SUB, _P1, _LN).swapaxes(1, 2).reshape(TOTAL)


def _from_phys(o):
    return o.reshape(_P0, _P1, _SUB, _LN).swapaxes(1, 2).reshape(200, 16384).T


def kernel(x, index, weight):
    out = _run(_to_phys(x), _to_phys(index), weight)
    return _from_phys(out)


# speedup vs baseline: 1.0264x; 1.0264x over previous
import functools

import jax
import jax.numpy as jnp
from jax import lax
from jax.experimental import pallas as pl
from jax.experimental.pallas import tpu as pltpu
from jax.experimental.pallas import tpu_sc as plsc

TOTAL = 16384 * 200
WLEN = 100000
NW = 32
PER_TILE = TOTAL // NW
CHUNK = 4096
NCHUNK = PER_TILE // CHUNK
LANES = 16
G = 8


def _sc_body(x_hbm, idx_hbm, w_hbm, out_hbm,
             w_v, idx_v, x_v, out_v, w_sem, in_sem, out_sem):
    wid = lax.axis_index("s") * 2 + lax.axis_index("c")
    tile_base = wid * PER_TILE

    def start_in(ci):
        b = ci % 2
        base = tile_base + ci * CHUNK
        pltpu.make_async_copy(
            idx_hbm.at[pl.ds(base, CHUNK)], idx_v.at[b], in_sem.at[b]).start()
        pltpu.make_async_copy(
            x_hbm.at[pl.ds(base, CHUNK)], x_v.at[b], in_sem.at[b]).start()

    def wait_in(ci):
        b = ci % 2
        base = tile_base + ci * CHUNK
        pltpu.make_async_copy(
            idx_hbm.at[pl.ds(base, CHUNK)], idx_v.at[b], in_sem.at[b]).wait()
        pltpu.make_async_copy(
            x_hbm.at[pl.ds(base, CHUNK)], x_v.at[b], in_sem.at[b]).wait()

    def start_out(ci):
        b = ci % 2
        base = tile_base + ci * CHUNK
        pltpu.make_async_copy(
            out_v.at[b], out_hbm.at[pl.ds(base, CHUNK)], out_sem.at[b]).start()

    def wait_out(ci):
        b = ci % 2
        base = tile_base + ci * CHUNK
        pltpu.make_async_copy(
            out_v.at[b], out_hbm.at[pl.ds(base, CHUNK)], out_sem.at[b]).wait()

    pltpu.make_async_copy(w_hbm, w_v, w_sem).start()
    start_in(0)
    pltpu.make_async_copy(w_hbm, w_v, w_sem).wait()

    for ci in range(NCHUNK):
        b = ci % 2
        if ci + 1 < NCHUNK:
            start_in(ci + 1)
        wait_in(ci)
        if ci >= 2:
            wait_out(ci - 2)

        def load_group(i):
            base = i * (G * LANES)
            ivs = [idx_v[b, pl.ds(base + k * LANES, LANES)] for k in range(G)]
            ws = [plsc.load_gather(w_v, [iv]) for iv in ivs]
            xs = [x_v[b, pl.ds(base + k * LANES, LANES)] for k in range(G)]
            return ws, xs

        def store_group(i, ws, xs):
            base = i * (G * LANES)
            for k in range(G):
                out_v[b, pl.ds(base + k * LANES, LANES)] = ws[k] * xs[k]

        ngroups = CHUNK // (G * LANES)

        def inner(i, carry):
            ws, xs = carry
            nxt = load_group(i + 1)
            store_group(i, ws, xs)
            return nxt

        last = lax.fori_loop(0, ngroups - 1, inner, load_group(0))
        store_group(ngroups - 1, *last)
        start_out(ci)

    wait_out(NCHUNK - 2)
    wait_out(NCHUNK - 1)


@jax.jit
def _run(x_flat, idx_flat, weight):
    mesh = plsc.VectorSubcoreMesh(core_axis_name="c", subcore_axis_name="s")
    f = functools.partial(
        pl.kernel,
        mesh=mesh,
        out_type=jax.ShapeDtypeStruct((TOTAL,), jnp.float32),
        scratch_types=[
            pltpu.VMEM((WLEN,), jnp.float32),
            pltpu.VMEM((2, CHUNK), jnp.int32),
            pltpu.VMEM((2, CHUNK), jnp.float32),
            pltpu.VMEM((2, CHUNK), jnp.float32),
            pltpu.SemaphoreType.DMA,
            pltpu.SemaphoreType.DMA((2,)),
            pltpu.SemaphoreType.DMA((2,)),
        ],
        compiler_params=pltpu.CompilerParams(needs_layout_passes=False),
    )(_sc_body)
    return f(x_flat, idx_flat, weight)


_P0, _P1, _SUB, _LN = 25, 128, 8, 128


def _to_phys(a):
    return a.T.reshape(_P0, _SUB, _P1, _LN).swapaxes(1, 2).reshape(TOTAL)


def _from_phys(o):
    return o.reshape(_P0, _P1, _SUB, _LN).swapaxes(1, 2).reshape(200, 16384).T


def kernel(x, index, weight):
    out = _run(_to_phys(x), _to_phys(index), weight)
    return _from_phys(out)


# speedup vs baseline: 1.1392x; 1.1099x over previous
import functools

import jax
import jax.numpy as jnp
from jax import lax
from jax.experimental import pallas as pl
from jax.experimental.pallas import tpu as pltpu
from jax.experimental.pallas import tpu_sc as plsc

TOTAL = 16384 * 200
WLEN = 100000
NW = 32
PER_TILE = TOTAL // NW
CHUNK = 4096
NCHUNK = PER_TILE // CHUNK
LANES = 16
G = 8


def _sc_body(x_hbm, idx_hbm, w_hbm, out_hbm,
             w_v, w_sh, idx_v, x_v, out_v, w_sem, in_sem, out_sem):
    sid = lax.axis_index("s")
    wid = sid * 2 + lax.axis_index("c")
    tile_base = wid * PER_TILE

    def start_in(ci):
        b = ci % 2
        base = tile_base + ci * CHUNK
        pltpu.make_async_copy(
            idx_hbm.at[pl.ds(base, CHUNK)], idx_v.at[b], in_sem.at[b]).start()
        pltpu.make_async_copy(
            x_hbm.at[pl.ds(base, CHUNK)], x_v.at[b], in_sem.at[b]).start()

    def wait_in(ci):
        b = ci % 2
        base = tile_base + ci * CHUNK
        pltpu.make_async_copy(
            idx_hbm.at[pl.ds(base, CHUNK)], idx_v.at[b], in_sem.at[b]).wait()
        pltpu.make_async_copy(
            x_hbm.at[pl.ds(base, CHUNK)], x_v.at[b], in_sem.at[b]).wait()

    def start_out(ci):
        b = ci % 2
        base = tile_base + ci * CHUNK
        pltpu.make_async_copy(
            out_v.at[b], out_hbm.at[pl.ds(base, CHUNK)], out_sem.at[b]).start()

    def wait_out(ci):
        b = ci % 2
        base = tile_base + ci * CHUNK
        pltpu.make_async_copy(
            out_v.at[b], out_hbm.at[pl.ds(base, CHUNK)], out_sem.at[b]).wait()

    start_in(0)

    @pl.when(sid == 0)
    def _():
        pltpu.sync_copy(w_hbm, w_sh)

    plsc.subcore_barrier()
    pltpu.sync_copy(w_sh, w_v)

    for ci in range(NCHUNK):
        b = ci % 2
        if ci + 1 < NCHUNK:
            start_in(ci + 1)
        wait_in(ci)
        if ci >= 2:
            wait_out(ci - 2)

        def load_group(i):
            base = i * (G * LANES)
            ivs = [idx_v[b, pl.ds(base + k * LANES, LANES)] for k in range(G)]
            ws = [plsc.load_gather(w_v, [iv]) for iv in ivs]
            xs = [x_v[b, pl.ds(base + k * LANES, LANES)] for k in range(G)]
            return ws, xs

        def store_group(i, ws, xs):
            base = i * (G * LANES)
            for k in range(G):
                out_v[b, pl.ds(base + k * LANES, LANES)] = ws[k] * xs[k]

        ngroups = CHUNK // (G * LANES)

        def inner(i, carry):
            ws, xs = carry
            nxt = load_group(i + 1)
            store_group(i, ws, xs)
            return nxt

        last = lax.fori_loop(0, ngroups - 1, inner, load_group(0))
        store_group(ngroups - 1, *last)
        start_out(ci)

    wait_out(NCHUNK - 2)
    wait_out(NCHUNK - 1)


@jax.jit
def _run(x_flat, idx_flat, weight):
    mesh = plsc.VectorSubcoreMesh(core_axis_name="c", subcore_axis_name="s")
    f = functools.partial(
        pl.kernel,
        mesh=mesh,
        out_type=jax.ShapeDtypeStruct((TOTAL,), jnp.float32),
        scratch_types=[
            pltpu.VMEM((WLEN,), jnp.float32),
            pltpu.VMEM_SHARED((WLEN,), jnp.float32),
            pltpu.VMEM((2, CHUNK), jnp.int32),
            pltpu.VMEM((2, CHUNK), jnp.float32),
            pltpu.VMEM((2, CHUNK), jnp.float32),
            pltpu.SemaphoreType.DMA,
            pltpu.SemaphoreType.DMA((2,)),
            pltpu.SemaphoreType.DMA((2,)),
        ],
        compiler_params=pltpu.CompilerParams(needs_layout_passes=False),
    )(_sc_body)
    return f(x_flat, idx_flat, weight)


_P0, _P1, _SUB, _LN = 25, 128, 8, 128


def _to_phys(a):
    return a.T.reshape(_P0, _SUB, _P1, _LN).swapaxes(1, 2).reshape(TOTAL)


def _from_phys(o):
    return o.reshape(_P0, _P1, _SUB, _LN).swapaxes(1, 2).reshape(200, 16384).T


def kernel(x, index, weight):
    out = _run(_to_phys(x), _to_phys(index), weight)
    return _from_phys(out)


# speedup vs baseline: 1.1559x; 1.0147x over previous
import functools

import jax
import jax.numpy as jnp
from jax import lax
from jax.experimental import pallas as pl
from jax.experimental.pallas import tpu as pltpu
from jax.experimental.pallas import tpu_sc as plsc

TOTAL = 16384 * 200
WLEN = 100000
NW = 32
PER_TILE = TOTAL // NW
CHUNK = 5120
NCHUNK = PER_TILE // CHUNK
LANES = 16
G = 8


def _sc_body(x_hbm, idx_hbm, w_hbm, out_hbm,
             w_v, w_sh, idx_v, x_v, w_sem, in_sem, out_sem):
    sid = lax.axis_index("s")
    wid = sid * 2 + lax.axis_index("c")
    tile_base = wid * PER_TILE

    def start_in(ci):
        b = ci % 2
        base = tile_base + ci * CHUNK
        pltpu.make_async_copy(
            idx_hbm.at[pl.ds(base, CHUNK)], idx_v.at[b], in_sem.at[b]).start()
        pltpu.make_async_copy(
            x_hbm.at[pl.ds(base, CHUNK)], x_v.at[b], in_sem.at[b]).start()

    def wait_in(ci):
        b = ci % 2
        base = tile_base + ci * CHUNK
        pltpu.make_async_copy(
            idx_hbm.at[pl.ds(base, CHUNK)], idx_v.at[b], in_sem.at[b]).wait()
        pltpu.make_async_copy(
            x_hbm.at[pl.ds(base, CHUNK)], x_v.at[b], in_sem.at[b]).wait()

    def start_out(ci):
        b = ci % 2
        base = tile_base + ci * CHUNK
        pltpu.make_async_copy(
            x_v.at[b], out_hbm.at[pl.ds(base, CHUNK)], out_sem.at[b]).start()

    def wait_out(ci):
        b = ci % 2
        base = tile_base + ci * CHUNK
        pltpu.make_async_copy(
            x_v.at[b], out_hbm.at[pl.ds(base, CHUNK)], out_sem.at[b]).wait()

    start_in(0)

    @pl.when(sid == 0)
    def _():
        pltpu.sync_copy(w_hbm, w_sh)

    plsc.subcore_barrier()
    pltpu.sync_copy(w_sh, w_v)

    for ci in range(NCHUNK):
        b = ci % 2
        if ci >= 1:
            wait_out(ci - 1)
        if ci + 1 < NCHUNK:
            start_in(ci + 1)
        wait_in(ci)

        def load_group(i):
            base = i * (G * LANES)
            ivs = [idx_v[b, pl.ds(base + k * LANES, LANES)] for k in range(G)]
            ws = [plsc.load_gather(w_v, [iv]) for iv in ivs]
            xs = [x_v[b, pl.ds(base + k * LANES, LANES)] for k in range(G)]
            return ws, xs

        def store_group(i, ws, xs):
            base = i * (G * LANES)
            for k in range(G):
                x_v[b, pl.ds(base + k * LANES, LANES)] = ws[k] * xs[k]

        ngroups = CHUNK // (G * LANES)

        def inner(i, carry):
            ws, xs = carry
            nxt = load_group(i + 1)
            store_group(i, ws, xs)
            return nxt

        last = lax.fori_loop(0, ngroups - 1, inner, load_group(0))
        store_group(ngroups - 1, *last)
        start_out(ci)

    wait_out(NCHUNK - 1)


@jax.jit
def _run(x_flat, idx_flat, weight):
    mesh = plsc.VectorSubcoreMesh(core_axis_name="c", subcore_axis_name="s")
    f = functools.partial(
        pl.kernel,
        mesh=mesh,
        out_type=jax.ShapeDtypeStruct((TOTAL,), jnp.float32),
        scratch_types=[
            pltpu.VMEM((WLEN,), jnp.float32),
            pltpu.VMEM_SHARED((WLEN,), jnp.float32),
            pltpu.VMEM((2, CHUNK), jnp.int32),
            pltpu.VMEM((2, CHUNK), jnp.float32),
            pltpu.SemaphoreType.DMA,
            pltpu.SemaphoreType.DMA((2,)),
            pltpu.SemaphoreType.DMA((2,)),
        ],
        compiler_params=pltpu.CompilerParams(needs_layout_passes=False),
    )(_sc_body)
    return f(x_flat, idx_flat, weight)


_P0, _P1, _SUB, _LN = 25, 128, 8, 128


def _to_phys(a):
    return a.T.reshape(_P0, _SUB, _P1, _LN).swapaxes(1, 2).reshape(TOTAL)


def _from_phys(o):
    return o.reshape(_P0, _P1, _SUB, _LN).swapaxes(1, 2).reshape(200, 16384).T


def kernel(x, index, weight):
    out = _run(_to_phys(x), _to_phys(index), weight)
    return _from_phys(out)
